# baseline (speedup 1.0000x reference)
import functools

import jax
import jax.numpy as jnp
from jax import lax
from jax.experimental import pallas as pl
from jax.experimental.pallas import tpu as pltpu
from jax.experimental.pallas import tpu_sc as plsc

from jax._src import core as jax_core
from jax._src.state import primitives as state_primitives
from jax._src.state import types as state_types
from jax._src.pallas.mosaic import sc_lowering as _scl
from jax._src.lib.mlir import ir
from jax.experimental.mosaic.dialects import tpu as _tpu

B = 16384
D = 32
NC = 2
NS = 16
L = 16
NW = NC * NS
BPW = B // NW
CHUNK = 128
NCHUNK = BPW // CHUNK
CL = D * CHUNK
NBLK = BPW // L

V = 1000000
VTILES = (V + 127) // 128
WPT = VTILES * 1024
FLAT_WORDS = (D // 8) * WPT


_flat_gather_start_p = jax_core.Primitive("flat_gather_start")
_flat_gather_start_p.multiple_results = True
_flat_gather_start_p.is_effectful = lambda params: True
_flat_gather_wait_p = jax_core.Primitive("flat_gather_wait")
_flat_gather_wait_p.multiple_results = True
_flat_gather_wait_p.is_effectful = lambda params: True


def _flat_gather_abstract_eval(*flat_args, tree, nsrc):
    del flat_args, tree, nsrc
    return (), {
        state_types.ReadEffect(0),
        state_types.ReadEffect(1),
        state_types.WriteEffect(2),
    }


_flat_gather_start_p.def_effectful_abstract_eval(_flat_gather_abstract_eval)
_flat_gather_wait_p.def_effectful_abstract_eval(_flat_gather_abstract_eval)


def _bind_flat_gather(prim, src, idx, dst, sem, nsrc):
    parts = []
    for x in (src, idx, dst, sem):
        ref, transforms = state_primitives.get_ref_and_transforms(
            x, None, "flat_gather")
        parts.append((ref, transforms))
    refs = [p[0] for p in parts]
    t_flat, t_tree = jax.tree.flatten([p[1] for p in parts])
    prim.bind(*refs, *t_flat, tree=t_tree, nsrc=nsrc)


def _flat_gather_refs(ctx, flat_args, tree, nsrc):
    refs, t_flat = flat_args[:4], flat_args[4:]
    avals, t_avals_flat = ctx.avals_in[:4], ctx.avals_in[4:]
    transforms = jax.tree.unflatten(tree, t_flat)
    transforms_aval = jax.tree.unflatten(tree, t_avals_flat)
    del transforms_aval
    out = []
    for ref, aval, t in zip(refs, avals, transforms):
        m, _ = _scl._transform_ref(ref, aval, aval.shape, t)
        out.append(m)
    src_m, idx_m, dst_m, sem_m = out
    src_ty = ir.MemRefType(src_m.type)
    dst_ty = ir.MemRefType(dst_m.type)
    del dst_ty
    flat_ty = ir.MemRefType.get(
        [nsrc], src_ty.element_type,
        layout=ir.Attribute.parse("#tpu.tiled<(128),[1]>"),
        memory_space=src_ty.memory_space)
    src_flat = _tpu.reinterpret_cast(flat_ty, src_m)
    return src_flat, idx_m, dst_m, sem_m


@_scl.register_lowering_rule(_flat_gather_start_p)
def _flat_gather_start_lowering(ctx, *flat_args, tree, nsrc):
    src_m, idx_m, dst_m, sem_m = _flat_gather_refs(ctx, flat_args, tree, nsrc)
    _tpu.enqueue_indirect_dma(
        src_m, dst_m, idx_m, sem_m, add=False, offset_filter=None)
    return ()


@_scl.register_lowering_rule(_flat_gather_wait_p)
def _flat_gather_wait_lowering(ctx, *flat_args, tree, nsrc):
    src_m, idx_m, dst_m, sem_m = _flat_gather_refs(ctx, flat_args, tree, nsrc)
    del idx_m
    _tpu.wait_indirect_dma(sem_m, src_m, dst_m)
    return ()


def _rsqrt(x):
    i = lax.bitcast_convert_type(x, jnp.int32)
    i = jnp.int32(0x5F3759DF) - lax.shift_right_arithmetic(i, 1)
    y = lax.bitcast_convert_type(i, jnp.float32)
    for _ in range(3):
        y = y * (1.5 - 0.5 * x * y * y)
    return y


_mesh = plsc.VectorSubcoreMesh(core_axis_name="c", subcore_axis_name="s")


@functools.partial(
    pl.kernel,
    mesh=_mesh,
    compiler_params=pltpu.CompilerParams(needs_layout_passes=False),
    out_type=jax.ShapeDtypeStruct((B,), jnp.float32),
    scratch_types=[
        pltpu.VMEM((BPW,), jnp.int32),
        pltpu.VMEM((BPW,), jnp.int32),
        pltpu.VMEM((BPW * D,), jnp.int32),
        pltpu.VMEM((BPW * D,), jnp.int32),
        pltpu.VMEM((BPW * D,), jnp.float32),
        pltpu.VMEM((BPW * D,), jnp.float32),
        pltpu.VMEM((BPW,), jnp.float32),
        pltpu.SemaphoreType.DMA,
    ],
)
def _als_sc(usr_hbm, movie_hbm, ut_hbm, mt_hbm, out_hbm,
            uidx_s, midx_s, uoff_s, moff_s, uw_v, mw_v, outv, sem):
    wid = lax.axis_index("s") * NC + lax.axis_index("c")
    base = wid * BPW

    pltpu.sync_copy(usr_hbm.at[pl.ds(base, BPW)], uidx_s)
    pltpu.sync_copy(movie_hbm.at[pl.ds(base, BPW)], midx_s)

    def prep(c, carry):
        co = pl.multiple_of(c * CHUNK, CHUNK)
        so = pl.multiple_of(c * CL, CL)
        for v in range(CHUNK // L):
            ko = v * L
            iu = uidx_s[pl.ds(co + ko, L)]
            im = midx_s[pl.ds(co + ko, L)]
            tu = ((iu >> 7) << 10) + (iu & 127)
            tm = ((im >> 7) << 10) + (im & 127)
            for d in range(D):
                cd = (d // 8) * WPT + (d % 8) * 128
                uoff_s[pl.ds(so + d * CHUNK + ko, L)] = tu + cd
                moff_s[pl.ds(so + d * CHUNK + ko, L)] = tm + cd
        sl = pl.ds(so, CL)
        _bind_flat_gather(_flat_gather_start_p, ut_hbm, uoff_s.at[sl],
                          uw_v.at[sl], sem, FLAT_WORDS)
        _bind_flat_gather(_flat_gather_start_p, mt_hbm, moff_s.at[sl],
                          mw_v.at[sl], sem, FLAT_WORDS)
        return carry

    lax.fori_loop(0, NCHUNK, prep, 0)

    for c in range(NCHUNK):
        so = pl.multiple_of(c * CL, CL)
        sl = pl.ds(so, CL)
        _bind_flat_gather(_flat_gather_wait_p, ut_hbm, uoff_s.at[sl],
                          uw_v.at[sl], sem, FLAT_WORDS)
        _bind_flat_gather(_flat_gather_wait_p, mt_hbm, moff_s.at[sl],
                          mw_v.at[sl], sem, FLAT_WORDS)

        def block(j, carry, c=c):
            off16 = c * CHUNK + j * L
            sbase = c * CL + j * L
            dot = jnp.zeros((L,), jnp.float32)
            uu = jnp.zeros((L,), jnp.float32)
            mm = jnp.zeros((L,), jnp.float32)
            for d in range(D):
                pos = pl.multiple_of(sbase + d * CHUNK, L)
                uv = uw_v[pl.ds(pos, L)]
                mv = mw_v[pl.ds(pos, L)]
                dot = dot + uv * mv
                uu = uu + uv * uv
                mm = mm + mv * mv
            su = jnp.minimum(jnp.float32(1.0), _rsqrt(uu))
            sm = jnp.minimum(jnp.float32(1.0), _rsqrt(mm))
            mx = dot * su * sm
            outv[pl.ds(off16, L)] = 1.0 / (1.0 + jnp.exp(-mx))
            return carry

        lax.fori_loop(0, CHUNK // L, block, 0)

    pltpu.sync_copy(outv, out_hbm.at[pl.ds(base, BPW)])


def kernel(usr, movie, usr_emd, movie_emd):
    return _als_sc(usr, movie, usr_emd.T, movie_emd.T)


# speedup vs baseline: 1.0010x; 1.0010x over previous
import functools

import jax
import jax.numpy as jnp
from jax import lax
from jax.experimental import pallas as pl
from jax.experimental.pallas import tpu as pltpu
from jax.experimental.pallas import tpu_sc as plsc

from jax._src import core as jax_core
from jax._src.state import primitives as state_primitives
from jax._src.state import types as state_types
from jax._src.pallas.mosaic import sc_lowering as _scl
from jax._src.lib.mlir import ir
from jax.experimental.mosaic.dialects import tpu as _tpu

B = 16384
D = 32
NC = 2
NS = 16
L = 16
NW = NC * NS
BPW = B // NW
CHUNK = 128
NCHUNK = BPW // CHUNK
CL = D * CHUNK
NBLK = BPW // L

V = 1000000
VTILES = (V + 127) // 128
WPT = VTILES * 1024
FLAT_WORDS = (D // 8) * WPT


_flat_gather_start_p = jax_core.Primitive("flat_gather_start")
_flat_gather_start_p.multiple_results = True
_flat_gather_start_p.is_effectful = lambda params: True
_flat_gather_wait_p = jax_core.Primitive("flat_gather_wait")
_flat_gather_wait_p.multiple_results = True
_flat_gather_wait_p.is_effectful = lambda params: True


def _flat_gather_abstract_eval(*flat_args, tree, nsrc):
    del flat_args, tree, nsrc
    return (), {
        state_types.ReadEffect(0),
        state_types.ReadEffect(1),
        state_types.WriteEffect(2),
    }


_flat_gather_start_p.def_effectful_abstract_eval(_flat_gather_abstract_eval)
_flat_gather_wait_p.def_effectful_abstract_eval(_flat_gather_abstract_eval)


def _bind_flat_gather(prim, src, idx, dst, sem, nsrc):
    parts = []
    for x in (src, idx, dst, sem):
        ref, transforms = state_primitives.get_ref_and_transforms(
            x, None, "flat_gather")
        parts.append((ref, transforms))
    refs = [p[0] for p in parts]
    t_flat, t_tree = jax.tree.flatten([p[1] for p in parts])
    prim.bind(*refs, *t_flat, tree=t_tree, nsrc=nsrc)


def _flat_gather_refs(ctx, flat_args, tree, nsrc):
    refs, t_flat = flat_args[:4], flat_args[4:]
    avals, t_avals_flat = ctx.avals_in[:4], ctx.avals_in[4:]
    transforms = jax.tree.unflatten(tree, t_flat)
    transforms_aval = jax.tree.unflatten(tree, t_avals_flat)
    del transforms_aval
    out = []
    for ref, aval, t in zip(refs, avals, transforms):
        m, _ = _scl._transform_ref(ref, aval, aval.shape, t)
        out.append(m)
    src_m, idx_m, dst_m, sem_m = out
    src_ty = ir.MemRefType(src_m.type)
    dst_ty = ir.MemRefType(dst_m.type)
    del dst_ty
    flat_ty = ir.MemRefType.get(
        [nsrc], src_ty.element_type,
        layout=ir.Attribute.parse("#tpu.tiled<(128),[1]>"),
        memory_space=src_ty.memory_space)
    src_flat = _tpu.reinterpret_cast(flat_ty, src_m)
    return src_flat, idx_m, dst_m, sem_m


@_scl.register_lowering_rule(_flat_gather_start_p)
def _flat_gather_start_lowering(ctx, *flat_args, tree, nsrc):
    src_m, idx_m, dst_m, sem_m = _flat_gather_refs(ctx, flat_args, tree, nsrc)
    _tpu.enqueue_indirect_dma(
        src_m, dst_m, idx_m, sem_m, add=False, offset_filter=None)
    return ()


@_scl.register_lowering_rule(_flat_gather_wait_p)
def _flat_gather_wait_lowering(ctx, *flat_args, tree, nsrc):
    src_m, idx_m, dst_m, sem_m = _flat_gather_refs(ctx, flat_args, tree, nsrc)
    del idx_m
    _tpu.wait_indirect_dma(sem_m, src_m, dst_m)
    return ()


def _rsqrt(x):
    i = lax.bitcast_convert_type(x, jnp.int32)
    i = jnp.int32(0x5F3759DF) - lax.shift_right_arithmetic(i, 1)
    y = lax.bitcast_convert_type(i, jnp.float32)
    for _ in range(3):
        y = y * (1.5 - 0.5 * x * y * y)
    return y


_mesh = plsc.VectorSubcoreMesh(core_axis_name="c", subcore_axis_name="s")


@functools.partial(
    pl.kernel,
    mesh=_mesh,
    compiler_params=pltpu.CompilerParams(
        needs_layout_passes=False,
        disable_bounds_checks=True,
        skip_device_barrier=True,
    ),
    out_type=jax.ShapeDtypeStruct((B,), jnp.float32),
    scratch_types=[
        pltpu.VMEM((BPW,), jnp.int32),
        pltpu.VMEM((BPW,), jnp.int32),
        pltpu.VMEM((BPW * D,), jnp.int32),
        pltpu.VMEM((BPW * D,), jnp.int32),
        pltpu.VMEM((BPW * D,), jnp.float32),
        pltpu.VMEM((BPW * D,), jnp.float32),
        pltpu.VMEM((BPW,), jnp.float32),
        pltpu.SemaphoreType.DMA,
    ],
)
def _als_sc(usr_hbm, movie_hbm, ut_hbm, mt_hbm, out_hbm,
            uidx_s, midx_s, uoff_s, moff_s, uw_v, mw_v, outv, sem):
    wid = lax.axis_index("s") * NC + lax.axis_index("c")
    base = wid * BPW

    pltpu.sync_copy(usr_hbm.at[pl.ds(base, BPW)], uidx_s)
    pltpu.sync_copy(movie_hbm.at[pl.ds(base, BPW)], midx_s)

    def prep(c, carry):
        co = pl.multiple_of(c * CHUNK, CHUNK)
        so = pl.multiple_of(c * CL, CL)
        for v in range(CHUNK // L):
            ko = v * L
            iu = uidx_s[pl.ds(co + ko, L)]
            im = midx_s[pl.ds(co + ko, L)]
            tu = ((iu >> 7) << 10) + (iu & 127)
            tm = ((im >> 7) << 10) + (im & 127)
            for d in range(D):
                cd = (d // 8) * WPT + (d % 8) * 128
                uoff_s[pl.ds(so + d * CHUNK + ko, L)] = tu + cd
                moff_s[pl.ds(so + d * CHUNK + ko, L)] = tm + cd
        sl = pl.ds(so, CL)
        _bind_flat_gather(_flat_gather_start_p, ut_hbm, uoff_s.at[sl],
                          uw_v.at[sl], sem, FLAT_WORDS)
        _bind_flat_gather(_flat_gather_start_p, mt_hbm, moff_s.at[sl],
                          mw_v.at[sl], sem, FLAT_WORDS)
        return carry

    lax.fori_loop(0, NCHUNK, prep, 0)

    for c in range(NCHUNK):
        so = pl.multiple_of(c * CL, CL)
        sl = pl.ds(so, CL)
        _bind_flat_gather(_flat_gather_wait_p, ut_hbm, uoff_s.at[sl],
                          uw_v.at[sl], sem, FLAT_WORDS)
        _bind_flat_gather(_flat_gather_wait_p, mt_hbm, moff_s.at[sl],
                          mw_v.at[sl], sem, FLAT_WORDS)

        def block(j, carry, c=c):
            off16 = c * CHUNK + j * L
            sbase = c * CL + j * L
            dot = jnp.zeros((L,), jnp.float32)
            uu = jnp.zeros((L,), jnp.float32)
            mm = jnp.zeros((L,), jnp.float32)
            for d in range(D):
                pos = pl.multiple_of(sbase + d * CHUNK, L)
                uv = uw_v[pl.ds(pos, L)]
                mv = mw_v[pl.ds(pos, L)]
                dot = dot + uv * mv
                uu = uu + uv * uv
                mm = mm + mv * mv
            su = jnp.minimum(jnp.float32(1.0), _rsqrt(uu))
            sm = jnp.minimum(jnp.float32(1.0), _rsqrt(mm))
            mx = dot * su * sm
            outv[pl.ds(off16, L)] = 1.0 / (1.0 + jnp.exp(-mx))
            return carry

        lax.fori_loop(0, CHUNK // L, block, 0)

    pltpu.sync_copy(outv, out_hbm.at[pl.ds(base, BPW)])


def kernel(usr, movie, usr_emd, movie_emd):
    return _als_sc(usr, movie, usr_emd.T, movie_emd.T)
